# baseline (speedup 1.0000x reference)
import functools

import jax
import jax.numpy as jnp
from jax import lax
from jax.experimental import pallas as pl
from jax.experimental.pallas import tpu as pltpu
from jax.experimental.pallas import tpu_sc as plsc

T = 2048
D = 1024
E = 8
K = 2
F = 2048
TM = 128
S = 4096 + E * TM
NT = S // TM


def _fiota(shape, dim):
    return lax.broadcasted_iota(jnp.int32, shape, dim).astype(jnp.float32)

def _router_body(x_ref, wg_ref, pos_ref, wsrt_ref, tsrc_ref, tmap_ref,
                 xbf_ref):
    x = x_ref[...]
    xb = x.astype(jnp.bfloat16)
    xu = lax.bitcast_convert_type(xb, jnp.uint16).astype(jnp.int32)
    xbf_ref[...] = xu[:, :D // 2] | (xu[:, D // 2:] << 16)
    logits = jnp.dot(x, wg_ref[...], preferred_element_type=jnp.float32)
    m = jnp.max(logits, axis=1, keepdims=True)
    p = jnp.exp(logits - m)
    probs = p / jnp.sum(p, axis=1, keepdims=True)
    lane = _fiota( (T, E), 1)
    m1 = jnp.max(probs, axis=1, keepdims=True)
    i1 = jnp.min(jnp.where(probs == m1, lane, float(E)), axis=1, keepdims=True)
    masked = jnp.where(lane == i1, -1.0, probs)
    m2 = jnp.max(masked, axis=1, keepdims=True)
    i2 = jnp.min(jnp.where(masked == m2, lane, float(E)), axis=1, keepdims=True)

    oh1 = (lane == i1).astype(jnp.float32)
    oh2 = (lane == i2).astype(jnp.float32)

    counts = jnp.sum(oh1 + oh2, axis=0, keepdims=True)
    padded = jnp.ceil(counts * (1.0 / TM)) * TM
    r8 = _fiota( (E, E), 0)
    c8 = _fiota( (E, E), 1)
    su = (r8 < c8).astype(jnp.float32)
    offs = jnp.dot(padded, su, preferred_element_type=jnp.float32)
    ends = offs + padded

    rt = _fiota( (T, T), 0)
    ct = _fiota( (T, T), 1)
    ltri = (ct < rt).astype(jnp.float32)
    run = jnp.zeros((1, E), jnp.float32)
    pos_k = []
    for oh in (oh1, oh2):
        rk = jnp.dot(ltri, oh, preferred_element_type=jnp.float32) + run
        pos_k.append(jnp.sum(oh * (offs + rk), axis=1))
        run = run + jnp.sum(oh, axis=0, keepdims=True)
    pos0f, pos1f = pos_k
    pos_ref[0, :] = pos0f.astype(jnp.int32)
    pos_ref[1, :] = pos1f.astype(jnp.int32)
    for r in range(2, 8):
        pos_ref[r, :] = jnp.zeros((T,), jnp.int32)

    tvec = _fiota( (1, T), 1)
    w0r = jnp.reshape(m1, (1, T))
    w1r = jnp.reshape(m2, (1, T))
    for b in range(S // 512):
        sblk = _fiota( (1, 512), 1) + (512.0 * b)
        m0 = (jnp.reshape(pos0f, (T, 1)) == sblk).astype(jnp.float32)
        m1b = (jnp.reshape(pos1f, (T, 1)) == sblk).astype(jnp.float32)
        ts = jnp.dot(tvec, m0, preferred_element_type=jnp.float32) + \
             jnp.dot(tvec, m1b, preferred_element_type=jnp.float32)
        ws = jnp.dot(w0r, m0, preferred_element_type=jnp.float32) + \
             jnp.dot(w1r, m1b, preferred_element_type=jnp.float32)
        tsrc_ref[0, pl.ds(b * 512, 512)] = jnp.reshape(ts.astype(jnp.int32), (512,))
        wsrt_ref[0, pl.ds(b * 512, 512)] = jnp.reshape(ws, (512,))
    for r in range(1, 8):
        tsrc_ref[r, :] = jnp.zeros((S,), jnp.int32)
        wsrt_ref[r, :] = jnp.zeros((S,), jnp.float32)

    tl = _fiota( (1, 128), 1) * float(TM)
    acc = jnp.zeros((1, 128), jnp.float32)
    for e in range(E):
        acc = acc + (tl >= ends[:, e:e + 1]).astype(jnp.float32)
    tmap_ref[0, :] = jnp.minimum(acc, float(E - 1)).astype(jnp.int32)[0, :]
    for r in range(1, 8):
        tmap_ref[r, :] = jnp.zeros((128,), jnp.int32)


def _router(x2d, wg, interpret=False):
    return pl.pallas_call(
        _router_body,
        out_shape=(
            jax.ShapeDtypeStruct((8, T), jnp.int32),
            jax.ShapeDtypeStruct((8, S), jnp.float32),
            jax.ShapeDtypeStruct((8, S), jnp.int32),
            jax.ShapeDtypeStruct((8, 128), jnp.int32),
            jax.ShapeDtypeStruct((T, D // 2), jnp.int32),
        ),
        interpret=interpret,
    )(x2d, wg)


def _gemm_body(tmap_ref, xs_ref, w1_ref, w3_ref, w2_ref, ws_ref, y_ref):
    xi = xs_ref[...]
    xlo = lax.bitcast_convert_type(xi << 16, jnp.float32)
    xhi = lax.bitcast_convert_type(xi & jnp.int32(-65536), jnp.float32)
    w1 = w1_ref[...]
    w3 = w3_ref[...]
    h = (jnp.dot(xlo, w1[:D // 2], preferred_element_type=jnp.float32) +
         jnp.dot(xhi, w1[D // 2:], preferred_element_type=jnp.float32))
    g = (jnp.dot(xlo, w3[:D // 2], preferred_element_type=jnp.float32) +
         jnp.dot(xhi, w3[D // 2:], preferred_element_type=jnp.float32))
    act = h * jax.nn.sigmoid(h) * g
    y = jnp.dot(act, w2_ref[...], preferred_element_type=jnp.float32)
    rr = _fiota( (TM, TM), 0)
    cc = _fiota( (TM, TM), 1)
    diag = jnp.where(rr == cc, ws_ref[...], 0.0)
    y_ref[...] = jnp.dot(diag, y, preferred_element_type=jnp.float32)


def _gemm_half(tmap, xs_half, w1, w3, w2, wsrt_rows, off, y_prev=None):
    nt2 = NT // 2
    in_specs = [
        pl.BlockSpec((TM, D // 2), lambda i, m: (i, 0)),
        pl.BlockSpec((None, D, F), lambda i, m: (m[i + off], 0, 0)),
        pl.BlockSpec((None, D, F), lambda i, m: (m[i + off], 0, 0)),
        pl.BlockSpec((None, F, D), lambda i, m: (m[i + off], 0, 0)),
        pl.BlockSpec((None, 1, TM), lambda i, m: (i + off, 0, 0)),
    ]
    args = [tmap, xs_half, w1, w3, w2, wsrt_rows]
    aliases = {}
    if y_prev is not None:
        in_specs.append(pl.BlockSpec((TM, D), lambda i, m: (0, 0)))
        args.append(y_prev)
        aliases = {6: 0}
    grid_spec = pltpu.PrefetchScalarGridSpec(
        num_scalar_prefetch=1,
        grid=(nt2,),
        in_specs=in_specs,
        out_specs=pl.BlockSpec((TM, D), lambda i, m: (i + off, 0)),
    )
    body = _gemm_body if y_prev is None else _gemm_body_alias
    return pl.pallas_call(
        body,
        grid_spec=grid_spec,
        out_shape=jax.ShapeDtypeStruct((S, D), jnp.float32),
        input_output_aliases=aliases,
    )(*args)


def _gemm_body_alias(tmap_ref, xs_ref, w1_ref, w3_ref, w2_ref, ws_ref,
                     yprev_ref, y_ref):
    _gemm_body(tmap_ref, xs_ref, w1_ref, w3_ref, w2_ref, ws_ref, y_ref)


_SC_MESH = plsc.VectorSubcoreMesh(core_axis_name="c", subcore_axis_name="s")
_NW = 32
_SH = S // 2
_GCH = 16
_GN = _SH // _NW // _GCH
_CCH = 16
_CN = T // _NW // _CCH


def _make_dispatch_half():
    @functools.partial(
        pl.kernel, mesh=_SC_MESH,
        out_type=jax.ShapeDtypeStruct((_SH, D // 2), jnp.int32),
        scratch_types=[
            pltpu.VMEM((2, _GCH), jnp.int32),
            pltpu.VMEM((_GCH, D // 2), jnp.int32),
            pltpu.VMEM((_GCH, D // 2), jnp.int32),
            pltpu.SemaphoreType.DMA,
            pltpu.SemaphoreType.DMA,
            pltpu.SemaphoreType.DMA,
            pltpu.SemaphoreType.DMA,
        ],
    )
    def _disp(x_hbm, tsrc_hbm, xs_hbm, idx_v, rows0_v, rows1_v,
              gs0, gs1, ws0, ws1):
        wid = lax.axis_index("s") * 2 + lax.axis_index("c")
        base = wid * (_GN * _GCH)
        rows = (rows0_v, rows1_v)
        gsem = (gs0, gs1)
        wsem = (ws0, ws1)
        h_g = [None] * _GN
        h_w = [None] * _GN
        pltpu.sync_copy(tsrc_hbm.at[pl.ds(base, _GCH)], idx_v.at[0])
        h_g[0] = pltpu.async_copy(x_hbm.at[idx_v.at[0]], rows[0], gsem[0])
        for c in range(_GN):
            s = c % 2
            o = (c + 1) % 2
            if c + 1 < _GN:
                if c - 1 >= 0:
                    h_w[c - 1].wait()
                pltpu.sync_copy(
                    tsrc_hbm.at[pl.ds(base + (c + 1) * _GCH, _GCH)],
                    idx_v.at[o])
                h_g[c + 1] = pltpu.async_copy(x_hbm.at[idx_v.at[o]], rows[o],
                                              gsem[o])
            h_g[c].wait()
            h_w[c] = pltpu.async_copy(
                rows[s], xs_hbm.at[pl.ds(base + c * _GCH, _GCH)], wsem[s])
        h_w[_GN - 2].wait()
        h_w[_GN - 1].wait()
    return _disp


_sc_dispatch_a = _make_dispatch_half()
_sc_dispatch_b = _make_dispatch_half()


@functools.partial(
    pl.kernel, mesh=_SC_MESH,
    out_type=jax.ShapeDtypeStruct((T, D), jnp.float32),
    scratch_types=[
        pltpu.VMEM((2, _CCH), jnp.int32),
        pltpu.VMEM((2, _CCH), jnp.int32),
        pltpu.VMEM((_CCH, D), jnp.float32),
        pltpu.VMEM((_CCH, D), jnp.float32),
        pltpu.VMEM((_CCH, D), jnp.float32),
        pltpu.VMEM((_CCH, D), jnp.float32),
        pltpu.VMEM((_CCH, D), jnp.float32),
        pltpu.VMEM((_CCH, D), jnp.float32),
        pltpu.SemaphoreType.DMA,
        pltpu.SemaphoreType.DMA,
        pltpu.SemaphoreType.DMA,
        pltpu.SemaphoreType.DMA,
        pltpu.SemaphoreType.DMA,
        pltpu.SemaphoreType.DMA,
    ],
)
def _sc_combine(y_hbm, pos0_hbm, pos1_hbm, out_hbm,
                idx0_v, idx1_v, a0_v, a1_v, b0_v, b1_v, o0_v, o1_v,
                ga0, ga1, gb0, gb1, ws0, ws1):
    wid = lax.axis_index("s") * 2 + lax.axis_index("c")
    base = wid * (_CN * _CCH)
    arows = (a0_v, a1_v)
    brows = (b0_v, b1_v)
    orows = (o0_v, o1_v)
    gasem = (ga0, ga1)
    gbsem = (gb0, gb1)
    wsem = (ws0, ws1)
    h_a = [None] * _CN
    h_b = [None] * _CN
    h_w = [None] * _CN

    def issue(c, s):
        pltpu.sync_copy(pos0_hbm.at[pl.ds(base + c * _CCH, _CCH)],
                        idx0_v.at[s])
        pltpu.sync_copy(pos1_hbm.at[pl.ds(base + c * _CCH, _CCH)],
                        idx1_v.at[s])
        h_a[c] = pltpu.async_copy(y_hbm.at[idx0_v.at[s]], arows[s], gasem[s])
        h_b[c] = pltpu.async_copy(y_hbm.at[idx1_v.at[s]], brows[s], gbsem[s])

    issue(0, 0)
    for c in range(_CN):
        s = c % 2
        o = (c + 1) % 2
        if c + 1 < _CN:
            if c - 1 >= 0:
                h_w[c - 1].wait()
            issue(c + 1, o)
        h_a[c].wait()
        h_b[c].wait()
        av, bv, ov = arows[s], brows[s], orows[s]

        def body(r, _):
            for v in range(D // 16):
                ov[r, pl.ds(v * 16, 16)] = (av[r, pl.ds(v * 16, 16)] +
                                            bv[r, pl.ds(v * 16, 16)])
            return 0

        lax.fori_loop(0, _CCH, body, 0)
        h_w[c] = pltpu.async_copy(ov, out_hbm.at[pl.ds(base + c * _CCH,
                                                       _CCH)], wsem[s])
    h_w[_CN - 2].wait()
    h_w[_CN - 1].wait()


def kernel(hidden_states, Wg, W1, W3, W2):
    orig_shape = hidden_states.shape
    x2d = hidden_states.reshape(-1, D)
    pos8, wsrt8, tsrc8, tmap8, xbf = _router(x2d, Wg)
    tmap = tmap8[0, :NT]
    token_src = tsrc8[0]
    wsrt_rows = wsrt8[0].reshape(NT, 1, TM)

    xs_a = _sc_dispatch_a(xbf, token_src[:_SH])
    xs_b = _sc_dispatch_b(xbf, token_src[_SH:])
    y1 = _gemm_half(tmap, xs_a, W1, W3, W2, wsrt_rows, 0)
    y = _gemm_half(tmap, xs_b, W1, W3, W2, wsrt_rows, NT // 2, y_prev=y1)
    out = _sc_combine(y, pos8[0], pos8[1])
    return out.reshape(orig_shape)


# speedup vs baseline: 1.0124x; 1.0124x over previous
import functools

import jax
import jax.numpy as jnp
from jax import lax
from jax.experimental import pallas as pl
from jax.experimental.pallas import tpu as pltpu
from jax.experimental.pallas import tpu_sc as plsc

T = 2048
D = 1024
E = 8
K = 2
F = 2048
TM = 128
S = 4096 + E * TM
NT = S // TM


def _fiota(shape, dim):
    return lax.broadcasted_iota(jnp.int32, shape, dim).astype(jnp.float32)

def _router_body(x_ref, wg_ref, pos_ref, wsrt_ref, tsrc_ref, tmap_ref,
                 xbf_ref):
    x = x_ref[...]
    xb = x.astype(jnp.bfloat16)
    xu = lax.bitcast_convert_type(xb, jnp.uint16).astype(jnp.int32)
    xbf_ref[...] = xu[:, :D // 2] | (xu[:, D // 2:] << 16)
    logits = jnp.dot(x, wg_ref[...], preferred_element_type=jnp.float32)
    m = jnp.max(logits, axis=1, keepdims=True)
    p = jnp.exp(logits - m)
    probs = p / jnp.sum(p, axis=1, keepdims=True)
    lane = _fiota( (T, E), 1)
    m1 = jnp.max(probs, axis=1, keepdims=True)
    i1 = jnp.min(jnp.where(probs == m1, lane, float(E)), axis=1, keepdims=True)
    masked = jnp.where(lane == i1, -1.0, probs)
    m2 = jnp.max(masked, axis=1, keepdims=True)
    i2 = jnp.min(jnp.where(masked == m2, lane, float(E)), axis=1, keepdims=True)

    oh1 = (lane == i1).astype(jnp.float32)
    oh2 = (lane == i2).astype(jnp.float32)

    counts = jnp.sum(oh1 + oh2, axis=0, keepdims=True)
    padded = jnp.ceil(counts * (1.0 / TM)) * TM
    r8 = _fiota( (E, E), 0)
    c8 = _fiota( (E, E), 1)
    su = (r8 < c8).astype(jnp.float32)
    offs = jnp.dot(padded, su, preferred_element_type=jnp.float32)
    ends = offs + padded

    rt = _fiota( (T, T), 0)
    ct = _fiota( (T, T), 1)
    ltri = (ct < rt).astype(jnp.float32)
    run = jnp.zeros((1, E), jnp.float32)
    pos_k = []
    for oh in (oh1, oh2):
        rk = jnp.dot(ltri, oh, preferred_element_type=jnp.float32) + run
        pos_k.append(jnp.sum(oh * (offs + rk), axis=1))
        run = run + jnp.sum(oh, axis=0, keepdims=True)
    pos0f, pos1f = pos_k
    pos_ref[0, :] = pos0f.astype(jnp.int32)
    pos_ref[1, :] = pos1f.astype(jnp.int32)
    for r in range(2, 8):
        pos_ref[r, :] = jnp.zeros((T,), jnp.int32)

    tvec = _fiota( (1, T), 1)
    w0r = jnp.reshape(m1, (1, T))
    w1r = jnp.reshape(m2, (1, T))
    for b in range(S // 512):
        sblk = _fiota( (1, 512), 1) + (512.0 * b)
        m0 = (jnp.reshape(pos0f, (T, 1)) == sblk).astype(jnp.float32)
        m1b = (jnp.reshape(pos1f, (T, 1)) == sblk).astype(jnp.float32)
        ts = jnp.dot(tvec, m0, preferred_element_type=jnp.float32) + \
             jnp.dot(tvec, m1b, preferred_element_type=jnp.float32)
        ws = jnp.dot(w0r, m0, preferred_element_type=jnp.float32) + \
             jnp.dot(w1r, m1b, preferred_element_type=jnp.float32)
        tsrc_ref[0, pl.ds(b * 512, 512)] = jnp.reshape(ts.astype(jnp.int32), (512,))
        wsrt_ref[0, pl.ds(b * 512, 512)] = jnp.reshape(ws, (512,))
    for r in range(1, 8):
        tsrc_ref[r, :] = jnp.zeros((S,), jnp.int32)
        wsrt_ref[r, :] = jnp.zeros((S,), jnp.float32)

    tl = _fiota( (1, 128), 1) * float(TM)
    acc = jnp.zeros((1, 128), jnp.float32)
    for e in range(E):
        acc = acc + (tl >= ends[:, e:e + 1]).astype(jnp.float32)
    tmap_ref[0, :] = jnp.minimum(acc, float(E - 1)).astype(jnp.int32)[0, :]
    for r in range(1, 8):
        tmap_ref[r, :] = jnp.zeros((128,), jnp.int32)


def _router(x2d, wg, interpret=False):
    return pl.pallas_call(
        _router_body,
        out_shape=(
            jax.ShapeDtypeStruct((8, T), jnp.int32),
            jax.ShapeDtypeStruct((8, S), jnp.float32),
            jax.ShapeDtypeStruct((8, S), jnp.int32),
            jax.ShapeDtypeStruct((8, 128), jnp.int32),
            jax.ShapeDtypeStruct((T, D // 2), jnp.int32),
        ),
        interpret=interpret,
    )(x2d, wg)


def _gemm_body(tmap_ref, xs_ref, w1_ref, w3_ref, w2_ref, ws_ref, y_ref):
    xi = xs_ref[...]
    xlo = lax.bitcast_convert_type(xi << 16, jnp.float32)
    xhi = lax.bitcast_convert_type(xi & jnp.int32(-65536), jnp.float32)
    w1 = w1_ref[...]
    w3 = w3_ref[...]
    h = (jnp.dot(xlo, w1[:D // 2], preferred_element_type=jnp.float32) +
         jnp.dot(xhi, w1[D // 2:], preferred_element_type=jnp.float32))
    g = (jnp.dot(xlo, w3[:D // 2], preferred_element_type=jnp.float32) +
         jnp.dot(xhi, w3[D // 2:], preferred_element_type=jnp.float32))
    act = h * jax.nn.sigmoid(h) * g
    y = jnp.dot(act, w2_ref[...], preferred_element_type=jnp.float32)
    rr = _fiota( (TM, TM), 0)
    cc = _fiota( (TM, TM), 1)
    diag = jnp.where(rr == cc, ws_ref[...], 0.0)
    y_ref[...] = jnp.dot(diag, y, preferred_element_type=jnp.float32)


def _gemm_full(tmap, xs, w1, w3, w2, wsrt_rows):
    grid_spec = pltpu.PrefetchScalarGridSpec(
        num_scalar_prefetch=1,
        grid=(NT,),
        in_specs=[
            pl.BlockSpec((TM, D // 2), lambda i, m: (i, 0)),
            pl.BlockSpec((None, D, F), lambda i, m: (m[i], 0, 0)),
            pl.BlockSpec((None, D, F), lambda i, m: (m[i], 0, 0)),
            pl.BlockSpec((None, F, D), lambda i, m: (m[i], 0, 0)),
            pl.BlockSpec((None, 1, TM), lambda i, m: (i, 0, 0)),
        ],
        out_specs=pl.BlockSpec((TM, D), lambda i, m: (i, 0)),
    )
    return pl.pallas_call(
        _gemm_body,
        grid_spec=grid_spec,
        out_shape=jax.ShapeDtypeStruct((S, D), jnp.float32),
    )(tmap, xs, w1, w3, w2, wsrt_rows)


def _gemm_half(tmap, xs_half, w1, w3, w2, wsrt_rows, off, y_prev=None):
    nt2 = NT // 2
    in_specs = [
        pl.BlockSpec((TM, D // 2), lambda i, m: (i, 0)),
        pl.BlockSpec((None, D, F), lambda i, m: (m[i + off], 0, 0)),
        pl.BlockSpec((None, D, F), lambda i, m: (m[i + off], 0, 0)),
        pl.BlockSpec((None, F, D), lambda i, m: (m[i + off], 0, 0)),
        pl.BlockSpec((None, 1, TM), lambda i, m: (i + off, 0, 0)),
    ]
    args = [tmap, xs_half, w1, w3, w2, wsrt_rows]
    aliases = {}
    if y_prev is not None:
        in_specs.append(pl.BlockSpec((TM, D), lambda i, m: (0, 0)))
        args.append(y_prev)
        aliases = {6: 0}
    grid_spec = pltpu.PrefetchScalarGridSpec(
        num_scalar_prefetch=1,
        grid=(nt2,),
        in_specs=in_specs,
        out_specs=pl.BlockSpec((TM, D), lambda i, m: (i + off, 0)),
    )
    body = _gemm_body if y_prev is None else _gemm_body_alias
    return pl.pallas_call(
        body,
        grid_spec=grid_spec,
        out_shape=jax.ShapeDtypeStruct((S, D), jnp.float32),
        input_output_aliases=aliases,
    )(*args)


def _gemm_body_alias(tmap_ref, xs_ref, w1_ref, w3_ref, w2_ref, ws_ref,
                     yprev_ref, y_ref):
    _gemm_body(tmap_ref, xs_ref, w1_ref, w3_ref, w2_ref, ws_ref, y_ref)


_SC_MESH = plsc.VectorSubcoreMesh(core_axis_name="c", subcore_axis_name="s")
_NW = 32
_SH = S // 2
_GCH = 16
_GN = _SH // _NW // _GCH
_CCH = 16
_CN = T // _NW // _CCH


_DR = S // _NW // 2


@functools.partial(
    pl.kernel, mesh=_SC_MESH,
    out_type=jax.ShapeDtypeStruct((S, D // 2), jnp.int32),
    scratch_types=[
        pltpu.VMEM((2, _DR), jnp.int32),
        pltpu.VMEM((_DR, D // 2), jnp.int32),
        pltpu.VMEM((_DR, D // 2), jnp.int32),
        pltpu.SemaphoreType.DMA,
        pltpu.SemaphoreType.DMA,
        pltpu.SemaphoreType.DMA,
        pltpu.SemaphoreType.DMA,
    ],
)
def _sc_dispatch(x_hbm, tsrc_hbm, xs_hbm, idx_v, rows0_v, rows1_v,
                 gs0, gs1, ws0, ws1):
    wid = lax.axis_index("s") * 2 + lax.axis_index("c")
    base = wid * (2 * _DR)
    pltpu.sync_copy(tsrc_hbm.at[pl.ds(base, _DR)], idx_v.at[0])
    pltpu.sync_copy(tsrc_hbm.at[pl.ds(base + _DR, _DR)], idx_v.at[1])
    g0 = pltpu.async_copy(x_hbm.at[idx_v.at[0]], rows0_v, gs0)
    g1 = pltpu.async_copy(x_hbm.at[idx_v.at[1]], rows1_v, gs1)
    g0.wait()
    w0 = pltpu.async_copy(rows0_v, xs_hbm.at[pl.ds(base, _DR)], ws0)
    g1.wait()
    w1 = pltpu.async_copy(rows1_v, xs_hbm.at[pl.ds(base + _DR, _DR)], ws1)
    w0.wait()
    w1.wait()


@functools.partial(
    pl.kernel, mesh=_SC_MESH,
    out_type=jax.ShapeDtypeStruct((T, D), jnp.float32),
    scratch_types=[
        pltpu.VMEM((2, _CCH), jnp.int32),
        pltpu.VMEM((2, _CCH), jnp.int32),
        pltpu.VMEM((_CCH, D), jnp.float32),
        pltpu.VMEM((_CCH, D), jnp.float32),
        pltpu.VMEM((_CCH, D), jnp.float32),
        pltpu.VMEM((_CCH, D), jnp.float32),
        pltpu.VMEM((_CCH, D), jnp.float32),
        pltpu.VMEM((_CCH, D), jnp.float32),
        pltpu.SemaphoreType.DMA,
        pltpu.SemaphoreType.DMA,
        pltpu.SemaphoreType.DMA,
        pltpu.SemaphoreType.DMA,
        pltpu.SemaphoreType.DMA,
        pltpu.SemaphoreType.DMA,
    ],
)
def _sc_combine(y_hbm, pos0_hbm, pos1_hbm, out_hbm,
                idx0_v, idx1_v, a0_v, a1_v, b0_v, b1_v, o0_v, o1_v,
                ga0, ga1, gb0, gb1, ws0, ws1):
    wid = lax.axis_index("s") * 2 + lax.axis_index("c")
    base = wid * (_CN * _CCH)
    arows = (a0_v, a1_v)
    brows = (b0_v, b1_v)
    orows = (o0_v, o1_v)
    gasem = (ga0, ga1)
    gbsem = (gb0, gb1)
    wsem = (ws0, ws1)
    h_a = [None] * _CN
    h_b = [None] * _CN
    h_w = [None] * _CN

    def issue(c, s):
        pltpu.sync_copy(pos0_hbm.at[pl.ds(base + c * _CCH, _CCH)],
                        idx0_v.at[s])
        pltpu.sync_copy(pos1_hbm.at[pl.ds(base + c * _CCH, _CCH)],
                        idx1_v.at[s])
        h_a[c] = pltpu.async_copy(y_hbm.at[idx0_v.at[s]], arows[s], gasem[s])
        h_b[c] = pltpu.async_copy(y_hbm.at[idx1_v.at[s]], brows[s], gbsem[s])

    issue(0, 0)
    for c in range(_CN):
        s = c % 2
        o = (c + 1) % 2
        if c + 1 < _CN:
            if c - 1 >= 0:
                h_w[c - 1].wait()
            issue(c + 1, o)
        h_a[c].wait()
        h_b[c].wait()
        av, bv, ov = arows[s], brows[s], orows[s]

        def body(r, _):
            for v in range(D // 16):
                ov[r, pl.ds(v * 16, 16)] = (av[r, pl.ds(v * 16, 16)] +
                                            bv[r, pl.ds(v * 16, 16)])
            return 0

        lax.fori_loop(0, _CCH, body, 0)
        h_w[c] = pltpu.async_copy(ov, out_hbm.at[pl.ds(base + c * _CCH,
                                                       _CCH)], wsem[s])
    h_w[_CN - 2].wait()
    h_w[_CN - 1].wait()


def kernel(hidden_states, Wg, W1, W3, W2):
    orig_shape = hidden_states.shape
    x2d = hidden_states.reshape(-1, D)
    pos8, wsrt8, tsrc8, tmap8, xbf = _router(x2d, Wg)
    tmap = tmap8[0, :NT]
    token_src = tsrc8[0]
    wsrt_rows = wsrt8[0].reshape(NT, 1, TM)

    xs = _sc_dispatch(xbf, token_src)
    y = _gemm_full(tmap, xs, W1, W3, W2, wsrt_rows)
    out = _sc_combine(y, pos8[0], pos8[1])
    return out.reshape(orig_shape)


# speedup vs baseline: 1.2066x; 1.1919x over previous
import functools

import jax
import jax.numpy as jnp
from jax import lax
from jax.experimental import pallas as pl
from jax.experimental.pallas import tpu as pltpu
from jax.experimental.pallas import tpu_sc as plsc

T = 2048
D = 1024
E = 8
K = 2
F = 2048
TM = 128
S = 4096 + E * TM
NT = S // TM


def _fiota(shape, dim):
    return lax.broadcasted_iota(jnp.int32, shape, dim).astype(jnp.float32)

def _router_body(x_ref, wg_ref, pos_ref, wsrt_ref, tsrc_ref, tmap_ref,
                 xbf_ref):
    x = x_ref[...]
    xb = x.astype(jnp.bfloat16)
    xu = lax.bitcast_convert_type(xb, jnp.uint16).astype(jnp.int32)
    xbf_ref[...] = xu[:, :D // 2] | (xu[:, D // 2:] << 16)
    logits = jnp.dot(x, wg_ref[...], preferred_element_type=jnp.float32)
    m = jnp.max(logits, axis=1, keepdims=True)
    p = jnp.exp(logits - m)
    probs = p / jnp.sum(p, axis=1, keepdims=True)
    lane = _fiota( (T, E), 1)
    m1 = jnp.max(probs, axis=1, keepdims=True)
    i1 = jnp.min(jnp.where(probs == m1, lane, float(E)), axis=1, keepdims=True)
    masked = jnp.where(lane == i1, -1.0, probs)
    m2 = jnp.max(masked, axis=1, keepdims=True)
    i2 = jnp.min(jnp.where(masked == m2, lane, float(E)), axis=1, keepdims=True)

    oh1 = (lane == i1).astype(jnp.float32)
    oh2 = (lane == i2).astype(jnp.float32)

    counts = jnp.sum(oh1 + oh2, axis=0, keepdims=True)
    padded = jnp.ceil(counts * (1.0 / TM)) * TM
    r8 = _fiota( (E, E), 0)
    c8 = _fiota( (E, E), 1)
    su = (r8 < c8).astype(jnp.float32)
    offs = jnp.dot(padded, su, preferred_element_type=jnp.float32)
    ends = offs + padded

    rt = _fiota( (T, T), 0)
    ct = _fiota( (T, T), 1)
    ltri = (ct < rt).astype(jnp.float32)
    run = jnp.zeros((1, E), jnp.float32)
    pos_k = []
    for oh in (oh1, oh2):
        rk = jnp.dot(ltri, oh, preferred_element_type=jnp.float32) + run
        pos_k.append(jnp.sum(oh * (offs + rk), axis=1))
        run = run + jnp.sum(oh, axis=0, keepdims=True)
    pos0f, pos1f = pos_k
    pos_ref[0, :] = pos0f.astype(jnp.int32)
    pos_ref[1, :] = pos1f.astype(jnp.int32)
    for r in range(2, 8):
        pos_ref[r, :] = jnp.zeros((T,), jnp.int32)

    tvec = _fiota( (1, T), 1)
    w0r = jnp.reshape(m1, (1, T))
    w1r = jnp.reshape(m2, (1, T))
    for b in range(S // 512):
        sblk = _fiota( (1, 512), 1) + (512.0 * b)
        m0 = (jnp.reshape(pos0f, (T, 1)) == sblk).astype(jnp.float32)
        m1b = (jnp.reshape(pos1f, (T, 1)) == sblk).astype(jnp.float32)
        ts = jnp.dot(tvec, m0, preferred_element_type=jnp.float32) + \
             jnp.dot(tvec, m1b, preferred_element_type=jnp.float32)
        hit = jnp.sum(m0 + m1b, axis=0, keepdims=True)
        ts = ts + (1.0 - hit) * jnp.floor(sblk * 0.25)
        ws = jnp.dot(w0r, m0, preferred_element_type=jnp.float32) + \
             jnp.dot(w1r, m1b, preferred_element_type=jnp.float32)
        tsrc_ref[0, pl.ds(b * 512, 512)] = jnp.reshape(ts.astype(jnp.int32), (512,))
        wsrt_ref[0, pl.ds(b * 512, 512)] = jnp.reshape(ws, (512,))
    for r in range(1, 8):
        tsrc_ref[r, :] = jnp.zeros((S,), jnp.int32)
        wsrt_ref[r, :] = jnp.zeros((S,), jnp.float32)

    tl = _fiota( (1, 128), 1) * float(TM)
    acc = jnp.zeros((1, 128), jnp.float32)
    for e in range(E):
        acc = acc + (tl >= ends[:, e:e + 1]).astype(jnp.float32)
    tmap_ref[0, :] = jnp.minimum(acc, float(E - 1)).astype(jnp.int32)[0, :]
    for r in range(1, 8):
        tmap_ref[r, :] = jnp.zeros((128,), jnp.int32)


def _router(x2d, wg, interpret=False):
    return pl.pallas_call(
        _router_body,
        out_shape=(
            jax.ShapeDtypeStruct((8, T), jnp.int32),
            jax.ShapeDtypeStruct((8, S), jnp.float32),
            jax.ShapeDtypeStruct((8, S), jnp.int32),
            jax.ShapeDtypeStruct((8, 128), jnp.int32),
            jax.ShapeDtypeStruct((T, D // 2), jnp.int32),
        ),
        interpret=interpret,
    )(x2d, wg)


def _gemm_body(tmap_ref, xs_ref, w1_ref, w3_ref, w2_ref, ws_ref, y_ref):
    xi = xs_ref[...]
    xlo = lax.bitcast_convert_type(xi << 16, jnp.float32)
    xhi = lax.bitcast_convert_type(xi & jnp.int32(-65536), jnp.float32)
    w1 = w1_ref[...]
    w3 = w3_ref[...]
    h = (jnp.dot(xlo, w1[:D // 2], preferred_element_type=jnp.float32) +
         jnp.dot(xhi, w1[D // 2:], preferred_element_type=jnp.float32))
    g = (jnp.dot(xlo, w3[:D // 2], preferred_element_type=jnp.float32) +
         jnp.dot(xhi, w3[D // 2:], preferred_element_type=jnp.float32))
    act = h * jax.nn.sigmoid(h) * g
    y = jnp.dot(act, w2_ref[...], preferred_element_type=jnp.float32)
    rr = _fiota( (TM, TM), 0)
    cc = _fiota( (TM, TM), 1)
    diag = jnp.where(rr == cc, ws_ref[...], 0.0)
    y_ref[...] = jnp.dot(diag, y, preferred_element_type=jnp.float32)


def _gemm_full(tmap, xs, w1, w3, w2, wsrt_rows):
    grid_spec = pltpu.PrefetchScalarGridSpec(
        num_scalar_prefetch=1,
        grid=(NT,),
        in_specs=[
            pl.BlockSpec((TM, D // 2), lambda i, m: (i, 0)),
            pl.BlockSpec((None, D, F), lambda i, m: (m[i], 0, 0)),
            pl.BlockSpec((None, D, F), lambda i, m: (m[i], 0, 0)),
            pl.BlockSpec((None, F, D), lambda i, m: (m[i], 0, 0)),
            pl.BlockSpec((None, 1, TM), lambda i, m: (i, 0, 0)),
        ],
        out_specs=pl.BlockSpec((TM, D), lambda i, m: (i, 0)),
    )
    return pl.pallas_call(
        _gemm_body,
        grid_spec=grid_spec,
        out_shape=jax.ShapeDtypeStruct((S, D), jnp.float32),
    )(tmap, xs, w1, w3, w2, wsrt_rows)


def _gemm_half(tmap, xs_half, w1, w3, w2, wsrt_rows, off, y_prev=None):
    nt2 = NT // 2
    in_specs = [
        pl.BlockSpec((TM, D // 2), lambda i, m: (i, 0)),
        pl.BlockSpec((None, D, F), lambda i, m: (m[i + off], 0, 0)),
        pl.BlockSpec((None, D, F), lambda i, m: (m[i + off], 0, 0)),
        pl.BlockSpec((None, F, D), lambda i, m: (m[i + off], 0, 0)),
        pl.BlockSpec((None, 1, TM), lambda i, m: (i + off, 0, 0)),
    ]
    args = [tmap, xs_half, w1, w3, w2, wsrt_rows]
    aliases = {}
    if y_prev is not None:
        in_specs.append(pl.BlockSpec((TM, D), lambda i, m: (0, 0)))
        args.append(y_prev)
        aliases = {6: 0}
    grid_spec = pltpu.PrefetchScalarGridSpec(
        num_scalar_prefetch=1,
        grid=(nt2,),
        in_specs=in_specs,
        out_specs=pl.BlockSpec((TM, D), lambda i, m: (i + off, 0)),
    )
    body = _gemm_body if y_prev is None else _gemm_body_alias
    return pl.pallas_call(
        body,
        grid_spec=grid_spec,
        out_shape=jax.ShapeDtypeStruct((S, D), jnp.float32),
        input_output_aliases=aliases,
    )(*args)


def _gemm_body_alias(tmap_ref, xs_ref, w1_ref, w3_ref, w2_ref, ws_ref,
                     yprev_ref, y_ref):
    _gemm_body(tmap_ref, xs_ref, w1_ref, w3_ref, w2_ref, ws_ref, y_ref)


_SC_MESH = plsc.VectorSubcoreMesh(core_axis_name="c", subcore_axis_name="s")
_NW = 32
_SH = S // 2
_GCH = 16
_GN = _SH // _NW // _GCH
_CCH = 16
_CN = T // _NW // _CCH


_DR = S // _NW // 2


@functools.partial(
    pl.kernel, mesh=_SC_MESH,
    out_type=jax.ShapeDtypeStruct((S, D // 2), jnp.int32),
    scratch_types=[
        pltpu.VMEM((2, _DR), jnp.int32),
        pltpu.VMEM((_DR, D // 2), jnp.int32),
        pltpu.VMEM((_DR, D // 2), jnp.int32),
        pltpu.SemaphoreType.DMA,
        pltpu.SemaphoreType.DMA,
        pltpu.SemaphoreType.DMA,
        pltpu.SemaphoreType.DMA,
    ],
)
def _sc_dispatch(x_hbm, tsrc_hbm, xs_hbm, idx_v, rows0_v, rows1_v,
                 gs0, gs1, ws0, ws1):
    wid = lax.axis_index("s") * 2 + lax.axis_index("c")
    base = wid * (2 * _DR)
    pltpu.sync_copy(tsrc_hbm.at[pl.ds(base, _DR)], idx_v.at[0])
    pltpu.sync_copy(tsrc_hbm.at[pl.ds(base + _DR, _DR)], idx_v.at[1])
    g0 = pltpu.async_copy(x_hbm.at[idx_v.at[0]], rows0_v, gs0)
    g1 = pltpu.async_copy(x_hbm.at[idx_v.at[1]], rows1_v, gs1)
    g0.wait()
    w0 = pltpu.async_copy(rows0_v, xs_hbm.at[pl.ds(base, _DR)], ws0)
    g1.wait()
    w1 = pltpu.async_copy(rows1_v, xs_hbm.at[pl.ds(base + _DR, _DR)], ws1)
    w0.wait()
    w1.wait()


@functools.partial(
    pl.kernel, mesh=_SC_MESH,
    out_type=jax.ShapeDtypeStruct((T, D), jnp.float32),
    scratch_types=[
        pltpu.VMEM((2, _CCH), jnp.int32),
        pltpu.VMEM((2, _CCH), jnp.int32),
        pltpu.VMEM((_CCH, D), jnp.float32),
        pltpu.VMEM((_CCH, D), jnp.float32),
        pltpu.VMEM((_CCH, D), jnp.float32),
        pltpu.VMEM((_CCH, D), jnp.float32),
        pltpu.VMEM((_CCH, D), jnp.float32),
        pltpu.VMEM((_CCH, D), jnp.float32),
        pltpu.SemaphoreType.DMA,
        pltpu.SemaphoreType.DMA,
        pltpu.SemaphoreType.DMA,
        pltpu.SemaphoreType.DMA,
        pltpu.SemaphoreType.DMA,
        pltpu.SemaphoreType.DMA,
    ],
)
def _sc_combine(y_hbm, pos0_hbm, pos1_hbm, out_hbm,
                idx0_v, idx1_v, a0_v, a1_v, b0_v, b1_v, o0_v, o1_v,
                ga0, ga1, gb0, gb1, ws0, ws1):
    wid = lax.axis_index("s") * 2 + lax.axis_index("c")
    base = wid * (_CN * _CCH)
    arows = (a0_v, a1_v)
    brows = (b0_v, b1_v)
    orows = (o0_v, o1_v)
    gasem = (ga0, ga1)
    gbsem = (gb0, gb1)
    wsem = (ws0, ws1)
    h_a = [None] * _CN
    h_b = [None] * _CN
    h_w = [None] * _CN

    def issue(c, s):
        pltpu.sync_copy(pos0_hbm.at[pl.ds(base + c * _CCH, _CCH)],
                        idx0_v.at[s])
        pltpu.sync_copy(pos1_hbm.at[pl.ds(base + c * _CCH, _CCH)],
                        idx1_v.at[s])
        h_a[c] = pltpu.async_copy(y_hbm.at[idx0_v.at[s]], arows[s], gasem[s])
        h_b[c] = pltpu.async_copy(y_hbm.at[idx1_v.at[s]], brows[s], gbsem[s])

    issue(0, 0)
    for c in range(_CN):
        s = c % 2
        o = (c + 1) % 2
        if c + 1 < _CN:
            if c - 1 >= 0:
                h_w[c - 1].wait()
            issue(c + 1, o)
        h_a[c].wait()
        h_b[c].wait()
        av, bv, ov = arows[s], brows[s], orows[s]

        def body(r, _):
            for v in range(D // 16):
                ov[r, pl.ds(v * 16, 16)] = (av[r, pl.ds(v * 16, 16)] +
                                            bv[r, pl.ds(v * 16, 16)])
            return 0

        lax.fori_loop(0, _CCH, body, 0)
        h_w[c] = pltpu.async_copy(ov, out_hbm.at[pl.ds(base + c * _CCH,
                                                       _CCH)], wsem[s])
    h_w[_CN - 2].wait()
    h_w[_CN - 1].wait()


def kernel(hidden_states, Wg, W1, W3, W2):
    orig_shape = hidden_states.shape
    x2d = hidden_states.reshape(-1, D)
    pos8, wsrt8, tsrc8, tmap8, xbf = _router(x2d, Wg)
    tmap = tmap8[0, :NT]
    token_src = tsrc8[0]
    wsrt_rows = wsrt8[0].reshape(NT, 1, TM)

    xs = _sc_dispatch(xbf, token_src)
    y = _gemm_full(tmap, xs, W1, W3, W2, wsrt_rows)
    out = _sc_combine(y, pos8[0], pos8[1])
    return out.reshape(orig_shape)
